# baseline (speedup 1.0000x reference)
import functools

import jax
import jax.numpy as jnp
from jax import lax
from jax.experimental import pallas as pl
from jax.experimental.pallas import tpu as pltpu
from jax.experimental.pallas import tpu_sc as plsc

TABLE_ROWS = 8192
TABLE_COLS = 8192
N_LOOKUPS = 1048576

NC = 2
NS = 16
NW = NC * NS
L = 16

N_PER_W = N_LOOKUPS // NW
CHUNK = 4096
N_SUPER = N_PER_W // CHUNK
G = 128
K = CHUNK // G

ROWS_PER_W = TABLE_ROWS // NW


def _flatten_body(tab_hbm, flat_hbm, sem):
    wid = lax.axis_index("s") * NC + lax.axis_index("c")
    row0 = wid * ROWS_PER_W

    def fire(r, _):
        row = row0 + r
        pltpu.async_copy(
            tab_hbm.at[row],
            flat_hbm.at[pl.ds(row * TABLE_COLS, TABLE_COLS)],
            sem,
        )
        return 0

    lax.fori_loop(0, ROWS_PER_W, fire, 0)
    pltpu.make_async_copy(
        flat_hbm.at[pl.ds(0, ROWS_PER_W * TABLE_COLS)],
        flat_hbm.at[pl.ds(row0 * TABLE_COLS, ROWS_PER_W * TABLE_COLS)],
        sem,
    ).wait()


def _gather_body(idx_hbm, flat_hbm, out_hbm, idx_stage, fidx, outbuf, sem):
    wid = lax.axis_index("s") * NC + lax.axis_index("c")
    lane = lax.iota(jnp.int32, L)

    def super_chunk(s, _):
        base = wid * N_PER_W + s * CHUNK
        pltpu.sync_copy(idx_hbm.at[pl.ds(base * 2, 2 * CHUNK)], idx_stage)

        def fcomp(j, _):
            ev = lane * 2 + j * (2 * L)
            r = plsc.load_gather(idx_stage, [ev])
            c = plsc.load_gather(idx_stage, [ev + 1])
            fidx[pl.ds(j * L, L)] = (r << 13) + c
            return 0

        lax.fori_loop(0, CHUNK // L, fcomp, 0)

        def fire(k, _):
            pltpu.async_copy(
                flat_hbm.at[fidx.at[pl.ds(k * G, G)]],
                outbuf.at[pl.ds(k * G, G)],
                sem,
            )
            return 0

        lax.fori_loop(0, K, fire, 0)
        pltpu.make_async_copy(flat_hbm.at[pl.ds(0, CHUNK)], outbuf, sem).wait()

        pltpu.sync_copy(outbuf, out_hbm.at[pl.ds(base, CHUNK)])
        return 0

    lax.fori_loop(0, N_SUPER, super_chunk, 0)


@jax.jit
def _run(indices, train_table):
    idx_flat = indices.reshape(2 * N_LOOKUPS)
    mesh = plsc.VectorSubcoreMesh(core_axis_name="c", subcore_axis_name="s")

    flatten = functools.partial(
        pl.kernel,
        mesh=mesh,
        out_type=jax.ShapeDtypeStruct((TABLE_ROWS * TABLE_COLS,), jnp.float32),
        scratch_types=[pltpu.SemaphoreType.DMA],
        compiler_params=pltpu.CompilerParams(needs_layout_passes=False),
    )(_flatten_body)
    tab_lin = flatten(train_table)

    gather = functools.partial(
        pl.kernel,
        mesh=mesh,
        out_type=jax.ShapeDtypeStruct((N_LOOKUPS,), jnp.float32),
        scratch_types=[
            pltpu.VMEM((2 * CHUNK,), jnp.int32),
            pltpu.VMEM((CHUNK,), jnp.int32),
            pltpu.VMEM((CHUNK,), jnp.float32),
            pltpu.SemaphoreType.DMA,
        ],
        compiler_params=pltpu.CompilerParams(needs_layout_passes=False),
    )(_gather_body)
    return gather(idx_flat, tab_lin)


def kernel(indices, train_table):
    return _run(indices.astype(jnp.int32), train_table)
